# baseline (speedup 1.0000x reference)
import functools

import numpy as np
import jax
import jax.numpy as jnp
from jax.experimental import pallas as pl
from jax.experimental.pallas import tpu as pltpu


def _parse(s):
    out = []
    for tok in s.split("+"):
        tok = tok.strip()
        mul, ir = tok.split("x")
        out.append((int(mul), int(ir[:-1]), 1 if ir[-1] == "e" else -1))
    return out


def _dim(irreps):
    return sum(mul * (2 * l + 1) for mul, l, _ in irreps)


def _round_up(x, m):
    return ((x + m - 1) // m) * m


def _compact_structure(ir1_s, ir2_s, iro_s):
    ir1, ir2, iro = _parse(ir1_s), _parse(ir2_s), _parse(iro_s)
    d1, d2, do = _dim(ir1), _dim(ir2), _dim(iro)

    offo, o = [], 0
    for mul, l, _ in iro:
        offo.append(o)
        o += mul * (2 * l + 1)

    instructions = []
    for i1, (m1, l1, p1) in enumerate(ir1):
        for i2, (m2, l2, p2) in enumerate(ir2):
            for io, (mo, lo, po) in enumerate(iro):
                if po == p1 * p2 and abs(l1 - l2) <= lo <= l1 + l2:
                    instructions.append((i1, i2, io))

    P = sum(ir1[i1][0] * ir2[i2][0] * iro[io][0] for i1, i2, io in instructions)
    p_pad = _round_up(P, 128)

    idx_tb, idx_p, idx_k = [], [], []
    p_off = 0
    for i1, i2, io in instructions:
        mul1 = ir1[i1][0]
        mul2 = ir2[i2][0]
        mulo, lo, _ = iro[io]
        ddo = 2 * lo + 1
        for u in range(mul1):
            for v in range(mul2):
                for w in range(mulo):
                    p = p_off + (u * mul2 + v) * mulo + w
                    k0 = offo[io] + w * ddo
                    for mo in range(ddo):
                        idx_tb.append((k0 + mo) * p_pad + p)
                        idx_p.append(p)
                        idx_k.append(k0 + mo)
        p_off += mul1 * mul2 * mulo

    order = np.argsort(np.asarray(idx_tb), kind="stable")
    idx_tb = np.asarray(idx_tb, np.int32)[order]
    idx_p = np.asarray(idx_p, np.int32)[order]
    idx_k = np.asarray(idx_k, np.int32)[order]

    C = idx_tb.shape[0]
    c_pad = _round_up(C, 128)
    e4c = np.zeros((c_pad, do), np.float32)
    e4c[np.arange(C), idx_k] = 1.0
    return dict(idx_tb=idx_tb, idx_p=idx_p, e4c=e4c, C=C, c_pad=c_pad,
                d1=d1, d2=d2, do=do)


_SEQ = [
    "16x0e",
    "16x0e + 2x1o + 2x2e",
    "16x0e + 2x1o + 2x1e + 2x2e + 2x2o",
    "1x0e + 1x0o + 1x1e + 1x1o + 1x2e + 1x2o + 1x3e + 1x3o",
]
_SH_IRREPS = "1x0e + 1x1o + 1x2e"
_STRUCT = [_compact_structure(_SEQ[i], _SH_IRREPS, _SEQ[i + 1]) for i in range(3)]

_TILE_E = 512


def _node_linear_kernel(x_ref, w_ref, b_ref, o_ref):
    o_ref[...] = (jnp.dot(x_ref[...], w_ref[...],
                          preferred_element_type=jnp.float32) + b_ref[...])


def _node_linear(x, w, b):
    n, din = x.shape
    dout = w.shape[1]
    tile = 2048
    while n % tile:
        tile //= 2
    return pl.pallas_call(
        _node_linear_kernel,
        out_shape=jax.ShapeDtypeStruct((n, dout), jnp.float32),
        grid=(n // tile,),
        in_specs=[pl.BlockSpec((tile, din), lambda i: (i, 0)),
                  pl.BlockSpec((din, dout), lambda i: (0, 0)),
                  pl.BlockSpec((1, dout), lambda i: (0, 0))],
        out_specs=pl.BlockSpec((tile, dout), lambda i: (i, 0)),
        compiler_params=pltpu.CompilerParams(
            dimension_semantics=("parallel",)),
    )(x, w, b)


def _tp_kernel(x1_ref, sh_ref, ea_ref,
               w1_ref, b1_ref, w2c_ref, b2c_ref,
               e1_ref, e2_ref, tbc_ref, e4c_ref,
               o_ref):
    f32 = jnp.float32
    bf16 = jnp.bfloat16
    h = jnp.dot(ea_ref[...], w1_ref[...], preferred_element_type=f32) + b1_ref[...]
    h = jnp.where(h > 20.0, h, jnp.log1p(jnp.exp(jnp.minimum(h, 20.0))))
    we = jnp.dot(h, w2c_ref[...], preferred_element_type=f32) + b2c_ref[...]

    x1e = jnp.dot(x1_ref[...], e1_ref[...], preferred_element_type=f32)
    she = jnp.dot(sh_ref[...], e2_ref[...], preferred_element_type=f32)
    z = x1e * she

    y = jnp.dot(z.astype(bf16), tbc_ref[...], preferred_element_type=f32)
    o_ref[...] = jnp.dot(we * y, e4c_ref[...], preferred_element_type=f32)


def _tp_layer(x1, sh, ea, w1, b1, w2c, b2c, e1, e2, tbc, e4c, do, tile_e):
    e_pad, d1 = x1.shape
    d2 = sh.shape[1]
    ed = ea.shape[1]

    def edge_map(i):
        return (i, 0)

    def const_map(i):
        return (0, 0)

    in_specs = [
        pl.BlockSpec((tile_e, d1), edge_map),
        pl.BlockSpec((tile_e, d2), edge_map),
        pl.BlockSpec((tile_e, ed), edge_map),
        pl.BlockSpec(w1.shape, const_map),
        pl.BlockSpec(b1.shape, const_map),
        pl.BlockSpec(w2c.shape, const_map),
        pl.BlockSpec(b2c.shape, const_map),
        pl.BlockSpec(e1.shape, const_map),
        pl.BlockSpec(e2.shape, const_map),
        pl.BlockSpec(tbc.shape, const_map),
        pl.BlockSpec(e4c.shape, const_map),
    ]
    return pl.pallas_call(
        _tp_kernel,
        out_shape=jax.ShapeDtypeStruct((e_pad, do), jnp.float32),
        grid=(e_pad // tile_e,),
        in_specs=in_specs,
        out_specs=pl.BlockSpec((tile_e, do), edge_map),
        compiler_params=pltpu.CompilerParams(
            dimension_semantics=("parallel",),
            vmem_limit_bytes=96 * 1024 * 1024),
    )(x1, sh, ea, w1, b1, w2c, b2c, e1, e2, tbc, e4c)


def _sph_harm(vec):
    import math
    r = jnp.linalg.norm(vec, axis=-1, keepdims=True)
    v = vec / jnp.maximum(r, 1e-12)
    x, y, z = v[..., 0], v[..., 1], v[..., 2]
    s3 = math.sqrt(3.0)
    sh0 = jnp.ones_like(x)[..., None]
    sh1 = s3 * jnp.stack([x, y, z], axis=-1)
    sh2 = math.sqrt(5.0) * jnp.stack(
        [s3 * x * z,
         s3 * x * y,
         y * y - 0.5 * (x * x + z * z),
         s3 * y * z,
         0.5 * s3 * (z * z - x * x)], axis=-1)
    return jnp.concatenate([sh0, sh1, sh2], axis=-1).astype(jnp.float32)


def kernel(node_feature, edge_vec, edge_feature, edge_index, node_w, node_b,
           w1_0, b1_0, w2_0, b2_0, E1_0, E2_0, TB_0, E4_0,
           w1_1, b1_1, w2_1, b2_1, E1_1, E2_1, TB_1, E4_1,
           w1_2, b1_2, w2_2, b2_2, E1_2, E2_2, TB_2, E4_2):
    layers = [
        (w1_0, b1_0, w2_0, b2_0, E1_0, E2_0, TB_0),
        (w1_1, b1_1, w2_1, b2_1, E1_1, E2_1, TB_1),
        (w1_2, b1_2, w2_2, b2_2, E1_2, E2_2, TB_2),
    ]
    edge_src, edge_dst = edge_index[0], edge_index[1]
    n_nodes = node_feature.shape[0]
    n_edges = edge_vec.shape[0]
    e_pad = _round_up(n_edges, _TILE_E)
    pad = e_pad - n_edges

    sh = _sph_harm(edge_vec)
    if pad:
        sh = jnp.pad(sh, ((0, pad), (0, 0)))
        edge_feature = jnp.pad(edge_feature, ((0, pad), (0, 0)))
        edge_dst = jnp.pad(edge_dst, (0, pad))

    nf = _node_linear(node_feature, node_w, node_b)

    cnt = jnp.zeros((n_nodes,), jnp.float32).at[edge_src].add(1.0)
    inv = 1.0 / jnp.maximum(cnt, 1.0)

    bf16 = jnp.bfloat16
    sh_b = sh
    ea_b = edge_feature

    for i, (w1, b1, w2, b2, e1, e2, tb) in enumerate(layers):
        st = _STRUCT[i]
        c_extra = st["c_pad"] - st["C"]
        tbc = jnp.take(tb, jnp.asarray(st["idx_tb"]), axis=1)
        w2c = jnp.take(w2, jnp.asarray(st["idx_p"]), axis=1)
        b2c = jnp.take(b2, jnp.asarray(st["idx_p"]), axis=1)
        if c_extra:
            tbc = jnp.pad(tbc, ((0, 0), (0, c_extra)))
            w2c = jnp.pad(w2c, ((0, 0), (0, c_extra)))
            b2c = jnp.pad(b2c, ((0, 0), (0, c_extra)))
        e4c = jnp.asarray(st["e4c"].astype(np.float32))

        x1 = jnp.take(nf, edge_dst, axis=0)
        tp = _tp_layer(x1, sh_b, ea_b, w1, b1,
                       w2c, b2c, e1, e2, tbc.astype(bf16),
                       e4c, st["do"], _TILE_E)[:n_edges]
        summed = jnp.zeros((n_nodes, st["do"]), jnp.float32).at[edge_src].add(tp)
        out = summed * inv[:, None]
        if i == 0:
            out = out + jnp.pad(nf, ((0, 0), (0, st["do"] - nf.shape[1])))
        nf = out
    return nf


# speedup vs baseline: 1.5138x; 1.5138x over previous
import functools

import numpy as np
import jax
import jax.numpy as jnp
from jax.experimental import pallas as pl
from jax.experimental.pallas import tpu as pltpu


def _parse(s):
    out = []
    for tok in s.split("+"):
        tok = tok.strip()
        mul, ir = tok.split("x")
        out.append((int(mul), int(ir[:-1]), 1 if ir[-1] == "e" else -1))
    return out


def _dim(irreps):
    return sum(mul * (2 * l + 1) for mul, l, _ in irreps)


def _round_up(x, m):
    return ((x + m - 1) // m) * m


def _compact_structure(ir1_s, ir2_s, iro_s):
    ir1, ir2, iro = _parse(ir1_s), _parse(ir2_s), _parse(iro_s)
    d1, d2, do = _dim(ir1), _dim(ir2), _dim(iro)

    offo, o = [], 0
    for mul, l, _ in iro:
        offo.append(o)
        o += mul * (2 * l + 1)

    instructions = []
    for i1, (m1, l1, p1) in enumerate(ir1):
        for i2, (m2, l2, p2) in enumerate(ir2):
            for io, (mo, lo, po) in enumerate(iro):
                if po == p1 * p2 and abs(l1 - l2) <= lo <= l1 + l2:
                    instructions.append((i1, i2, io))

    P = sum(ir1[i1][0] * ir2[i2][0] * iro[io][0] for i1, i2, io in instructions)
    p_pad = _round_up(P, 128)

    idx_tb, idx_p, idx_k = [], [], []
    p_off = 0
    for i1, i2, io in instructions:
        mul1 = ir1[i1][0]
        mul2 = ir2[i2][0]
        mulo, lo, _ = iro[io]
        ddo = 2 * lo + 1
        for u in range(mul1):
            for v in range(mul2):
                for w in range(mulo):
                    p = p_off + (u * mul2 + v) * mulo + w
                    k0 = offo[io] + w * ddo
                    for mo in range(ddo):
                        idx_tb.append((k0 + mo) * p_pad + p)
                        idx_p.append(p)
                        idx_k.append(k0 + mo)
        p_off += mul1 * mul2 * mulo

    order = np.argsort(np.asarray(idx_tb), kind="stable")
    idx_tb = np.asarray(idx_tb, np.int32)[order]
    idx_p = np.asarray(idx_p, np.int32)[order]
    idx_k = np.asarray(idx_k, np.int32)[order]

    C = idx_tb.shape[0]
    c_pad = _round_up(C, 128)
    e4c = np.zeros((c_pad, do), np.float32)
    e4c[np.arange(C), idx_k] = 1.0
    return dict(idx_tb=idx_tb, idx_p=idx_p, e4c=e4c, C=C, c_pad=c_pad,
                d1=d1, d2=d2, do=do)


_SEQ = [
    "16x0e",
    "16x0e + 2x1o + 2x2e",
    "16x0e + 2x1o + 2x1e + 2x2e + 2x2o",
    "1x0e + 1x0o + 1x1e + 1x1o + 1x2e + 1x2o + 1x3e + 1x3o",
]
_SH_IRREPS = "1x0e + 1x1o + 1x2e"
_STRUCT = [_compact_structure(_SEQ[i], _SH_IRREPS, _SEQ[i + 1]) for i in range(3)]

_TILE_E = 512


def _node_linear_kernel(x_ref, w_ref, b_ref, o_ref):
    o_ref[...] = (jnp.dot(x_ref[...], w_ref[...],
                          preferred_element_type=jnp.float32) + b_ref[...])


def _node_linear(x, w, b):
    n, din = x.shape
    dout = w.shape[1]
    tile = 2048
    while n % tile:
        tile //= 2
    return pl.pallas_call(
        _node_linear_kernel,
        out_shape=jax.ShapeDtypeStruct((n, dout), jnp.float32),
        grid=(n // tile,),
        in_specs=[pl.BlockSpec((tile, din), lambda i: (i, 0)),
                  pl.BlockSpec((din, dout), lambda i: (0, 0)),
                  pl.BlockSpec((1, dout), lambda i: (0, 0))],
        out_specs=pl.BlockSpec((tile, dout), lambda i: (i, 0)),
        compiler_params=pltpu.CompilerParams(
            dimension_semantics=("parallel",)),
    )(x, w, b)


def _tp_kernel(tile_e, dst_ref, nf_ref, sh_ref, ea_ref,
               w1_ref, b1_ref, w2c_ref, b2c_ref,
               e1_ref, e2_ref, tbc_ref, e4c_ref,
               o_ref, x1t_ref):
    f32 = jnp.float32
    bf16 = jnp.bfloat16

    for mi in range(tile_e):
        x1t_ref[mi, :] = nf_ref[dst_ref[0, 0, mi], 0, :]

    h = jnp.dot(ea_ref[...], w1_ref[...], preferred_element_type=f32) + b1_ref[...]
    h = jnp.where(h > 20.0, h, jnp.log1p(jnp.exp(jnp.minimum(h, 20.0))))
    we = jnp.dot(h, w2c_ref[...], preferred_element_type=f32) + b2c_ref[...]

    x1e = jnp.dot(x1t_ref[...], e1_ref[...], preferred_element_type=f32)
    she = jnp.dot(sh_ref[...], e2_ref[...], preferred_element_type=f32)
    z = x1e * she

    y = jnp.dot(z.astype(bf16), tbc_ref[...], preferred_element_type=f32)
    o_ref[...] = jnp.dot(we * y, e4c_ref[...], preferred_element_type=f32)


def _tp_layer(nf, dst3, sh, ea, w1, b1, w2c, b2c, e1, e2, tbc, e4c, do, tile_e):
    e_pad = sh.shape[0]
    d2 = sh.shape[1]
    ed = ea.shape[1]
    n_nodes, _, d1 = nf.shape

    def edge_map(i):
        return (i, 0)

    def const_map(i):
        return (0, 0)

    in_specs = [
        pl.BlockSpec((1, 1, tile_e), lambda i: (i, 0, 0),
                     memory_space=pltpu.SMEM),
        pl.BlockSpec((n_nodes, 1, d1), lambda i: (0, 0, 0)),
        pl.BlockSpec((tile_e, d2), edge_map),
        pl.BlockSpec((tile_e, ed), edge_map),
        pl.BlockSpec(w1.shape, const_map),
        pl.BlockSpec(b1.shape, const_map),
        pl.BlockSpec(w2c.shape, const_map),
        pl.BlockSpec(b2c.shape, const_map),
        pl.BlockSpec(e1.shape, const_map),
        pl.BlockSpec(e2.shape, const_map),
        pl.BlockSpec(tbc.shape, const_map),
        pl.BlockSpec(e4c.shape, const_map),
    ]
    return pl.pallas_call(
        functools.partial(_tp_kernel, tile_e),
        out_shape=jax.ShapeDtypeStruct((e_pad, do), jnp.float32),
        grid=(e_pad // tile_e,),
        in_specs=in_specs,
        out_specs=pl.BlockSpec((tile_e, do), edge_map),
        scratch_shapes=[pltpu.VMEM((tile_e, d1), jnp.float32)],
        compiler_params=pltpu.CompilerParams(
            dimension_semantics=("parallel",),
            vmem_limit_bytes=96 * 1024 * 1024),
    )(dst3, nf, sh, ea, w1, b1, w2c, b2c, e1, e2, tbc, e4c)


def _sph_harm(vec):
    import math
    r = jnp.linalg.norm(vec, axis=-1, keepdims=True)
    v = vec / jnp.maximum(r, 1e-12)
    x, y, z = v[..., 0], v[..., 1], v[..., 2]
    s3 = math.sqrt(3.0)
    sh0 = jnp.ones_like(x)[..., None]
    sh1 = s3 * jnp.stack([x, y, z], axis=-1)
    sh2 = math.sqrt(5.0) * jnp.stack(
        [s3 * x * z,
         s3 * x * y,
         y * y - 0.5 * (x * x + z * z),
         s3 * y * z,
         0.5 * s3 * (z * z - x * x)], axis=-1)
    return jnp.concatenate([sh0, sh1, sh2], axis=-1).astype(jnp.float32)


def kernel(node_feature, edge_vec, edge_feature, edge_index, node_w, node_b,
           w1_0, b1_0, w2_0, b2_0, E1_0, E2_0, TB_0, E4_0,
           w1_1, b1_1, w2_1, b2_1, E1_1, E2_1, TB_1, E4_1,
           w1_2, b1_2, w2_2, b2_2, E1_2, E2_2, TB_2, E4_2):
    layers = [
        (w1_0, b1_0, w2_0, b2_0, E1_0, E2_0, TB_0),
        (w1_1, b1_1, w2_1, b2_1, E1_1, E2_1, TB_1),
        (w1_2, b1_2, w2_2, b2_2, E1_2, E2_2, TB_2),
    ]
    edge_src, edge_dst = edge_index[0], edge_index[1]
    n_nodes = node_feature.shape[0]
    n_edges = edge_vec.shape[0]
    e_pad = _round_up(n_edges, _TILE_E)
    pad = e_pad - n_edges

    sh = _sph_harm(edge_vec)
    if pad:
        sh = jnp.pad(sh, ((0, pad), (0, 0)))
        edge_feature = jnp.pad(edge_feature, ((0, pad), (0, 0)))
        edge_dst = jnp.pad(edge_dst, (0, pad))

    nf = _node_linear(node_feature, node_w, node_b)

    cnt = jnp.zeros((n_nodes,), jnp.float32).at[edge_src].add(1.0)
    inv = 1.0 / jnp.maximum(cnt, 1.0)

    bf16 = jnp.bfloat16
    sh_b = sh
    ea_b = edge_feature
    dst3 = edge_dst.astype(jnp.int32).reshape(e_pad // _TILE_E, 1, _TILE_E)

    for i, (w1, b1, w2, b2, e1, e2, tb) in enumerate(layers):
        st = _STRUCT[i]
        c_extra = st["c_pad"] - st["C"]
        tbc = jnp.take(tb, jnp.asarray(st["idx_tb"]), axis=1)
        w2c = jnp.take(w2, jnp.asarray(st["idx_p"]), axis=1)
        b2c = jnp.take(b2, jnp.asarray(st["idx_p"]), axis=1)
        if c_extra:
            tbc = jnp.pad(tbc, ((0, 0), (0, c_extra)))
            w2c = jnp.pad(w2c, ((0, 0), (0, c_extra)))
            b2c = jnp.pad(b2c, ((0, 0), (0, c_extra)))
        e4c = jnp.asarray(st["e4c"].astype(np.float32))

        tp = _tp_layer(nf[:, None, :], dst3, sh_b, ea_b, w1, b1,
                       w2c, b2c, e1, e2, tbc.astype(bf16),
                       e4c, st["do"], _TILE_E)[:n_edges]
        summed = jnp.zeros((n_nodes, st["do"]), jnp.float32).at[edge_src].add(tp)
        out = summed * inv[:, None]
        if i == 0:
            out = out + jnp.pad(nf, ((0, 0), (0, st["do"] - nf.shape[1])))
        nf = out
    return nf


# speedup vs baseline: 1.8049x; 1.1923x over previous
import functools

import numpy as np
import jax
import jax.numpy as jnp
from jax.experimental import pallas as pl
from jax.experimental.pallas import tpu as pltpu


def _parse(s):
    out = []
    for tok in s.split("+"):
        tok = tok.strip()
        mul, ir = tok.split("x")
        out.append((int(mul), int(ir[:-1]), 1 if ir[-1] == "e" else -1))
    return out


def _dim(irreps):
    return sum(mul * (2 * l + 1) for mul, l, _ in irreps)


def _round_up(x, m):
    return ((x + m - 1) // m) * m


def _compact_structure(ir1_s, ir2_s, iro_s):
    ir1, ir2, iro = _parse(ir1_s), _parse(ir2_s), _parse(iro_s)
    d1, d2, do = _dim(ir1), _dim(ir2), _dim(iro)

    offo, o = [], 0
    for mul, l, _ in iro:
        offo.append(o)
        o += mul * (2 * l + 1)

    instructions = []
    for i1, (m1, l1, p1) in enumerate(ir1):
        for i2, (m2, l2, p2) in enumerate(ir2):
            for io, (mo, lo, po) in enumerate(iro):
                if po == p1 * p2 and abs(l1 - l2) <= lo <= l1 + l2:
                    instructions.append((i1, i2, io))

    P = sum(ir1[i1][0] * ir2[i2][0] * iro[io][0] for i1, i2, io in instructions)
    p_pad = _round_up(P, 128)

    idx_tb, idx_p, idx_k = [], [], []
    p_off = 0
    for i1, i2, io in instructions:
        mul1 = ir1[i1][0]
        mul2 = ir2[i2][0]
        mulo, lo, _ = iro[io]
        ddo = 2 * lo + 1
        for u in range(mul1):
            for v in range(mul2):
                for w in range(mulo):
                    p = p_off + (u * mul2 + v) * mulo + w
                    k0 = offo[io] + w * ddo
                    for mo in range(ddo):
                        idx_tb.append((k0 + mo) * p_pad + p)
                        idx_p.append(p)
                        idx_k.append(k0 + mo)
        p_off += mul1 * mul2 * mulo

    order = np.argsort(np.asarray(idx_tb), kind="stable")
    idx_tb = np.asarray(idx_tb, np.int32)[order]
    idx_p = np.asarray(idx_p, np.int32)[order]
    idx_k = np.asarray(idx_k, np.int32)[order]

    C = idx_tb.shape[0]
    c_pad = _round_up(C, 128)
    e4c = np.zeros((c_pad, do), np.float32)
    e4c[np.arange(C), idx_k] = 1.0
    return dict(idx_tb=idx_tb, idx_p=idx_p, e4c=e4c, C=C, c_pad=c_pad,
                d1=d1, d2=d2, do=do)


_SEQ = [
    "16x0e",
    "16x0e + 2x1o + 2x2e",
    "16x0e + 2x1o + 2x1e + 2x2e + 2x2o",
    "1x0e + 1x0o + 1x1e + 1x1o + 1x2e + 1x2o + 1x3e + 1x3o",
]
_SH_IRREPS = "1x0e + 1x1o + 1x2e"
_STRUCT = [_compact_structure(_SEQ[i], _SH_IRREPS, _SEQ[i + 1]) for i in range(3)]

_TILE_E = 512


def _node_linear_kernel(x_ref, w_ref, b_ref, o_ref):
    o_ref[...] = (jnp.dot(x_ref[...], w_ref[...],
                          preferred_element_type=jnp.float32) + b_ref[...])


def _node_linear(x, w, b):
    n, din = x.shape
    dout = w.shape[1]
    tile = 2048
    while n % tile:
        tile //= 2
    return pl.pallas_call(
        _node_linear_kernel,
        out_shape=jax.ShapeDtypeStruct((n, dout), jnp.float32),
        grid=(n // tile,),
        in_specs=[pl.BlockSpec((tile, din), lambda i: (i, 0)),
                  pl.BlockSpec((din, dout), lambda i: (0, 0)),
                  pl.BlockSpec((1, dout), lambda i: (0, 0))],
        out_specs=pl.BlockSpec((tile, dout), lambda i: (i, 0)),
        compiler_params=pltpu.CompilerParams(
            dimension_semantics=("parallel",)),
    )(x, w, b)


def _tp_kernel(tile_e, dst_ref, nf_ref, sh_ref, ea_ref,
               w1_ref, b1_ref, w2c_ref, b2c_ref,
               e1_ref, e2_ref, tbc_ref, e4c_ref,
               o_ref, x1t_ref):
    f32 = jnp.float32
    bf16 = jnp.bfloat16

    for mi in range(tile_e):
        x1t_ref[mi, :] = nf_ref[dst_ref[0, 0, mi], 0, :]

    h = jnp.dot(ea_ref[...], w1_ref[...], preferred_element_type=f32) + b1_ref[...]
    h = jnp.where(h > 20.0, h, jnp.log1p(jnp.exp(jnp.minimum(h, 20.0))))
    we = jnp.dot(h, w2c_ref[...], preferred_element_type=f32) + b2c_ref[...]

    x1e = jnp.dot(x1t_ref[...], e1_ref[...], preferred_element_type=f32)
    she = jnp.dot(sh_ref[...], e2_ref[...], preferred_element_type=f32)
    z = x1e * she

    y = jnp.dot(z.astype(bf16), tbc_ref[...], preferred_element_type=f32)
    o_ref[...] = jnp.dot(we * y, e4c_ref[...], preferred_element_type=f32)


def _tp_layer(nf, dst3, sh, ea, w1, b1, w2c, b2c, e1, e2, tbc, e4c, do, tile_e):
    e_pad = sh.shape[0]
    d2 = sh.shape[1]
    ed = ea.shape[1]
    n_nodes, _, d1 = nf.shape

    def edge_map(i):
        return (i, 0)

    def const_map(i):
        return (0, 0)

    in_specs = [
        pl.BlockSpec((1, 1, tile_e), lambda i: (i, 0, 0),
                     memory_space=pltpu.SMEM),
        pl.BlockSpec((n_nodes, 1, d1), lambda i: (0, 0, 0)),
        pl.BlockSpec((tile_e, d2), edge_map),
        pl.BlockSpec((tile_e, ed), edge_map),
        pl.BlockSpec(w1.shape, const_map),
        pl.BlockSpec(b1.shape, const_map),
        pl.BlockSpec(w2c.shape, const_map),
        pl.BlockSpec(b2c.shape, const_map),
        pl.BlockSpec(e1.shape, const_map),
        pl.BlockSpec(e2.shape, const_map),
        pl.BlockSpec(tbc.shape, const_map),
        pl.BlockSpec(e4c.shape, const_map),
    ]
    return pl.pallas_call(
        functools.partial(_tp_kernel, tile_e),
        out_shape=jax.ShapeDtypeStruct((e_pad, do), jnp.float32),
        grid=(e_pad // tile_e,),
        in_specs=in_specs,
        out_specs=pl.BlockSpec((tile_e, do), edge_map),
        scratch_shapes=[pltpu.VMEM((tile_e, d1), jnp.float32)],
        compiler_params=pltpu.CompilerParams(
            dimension_semantics=("parallel",),
            vmem_limit_bytes=96 * 1024 * 1024),
    )(dst3, nf, sh, ea, w1, b1, w2c, b2c, e1, e2, tbc, e4c)


def _sph_harm(vec):
    import math
    r = jnp.linalg.norm(vec, axis=-1, keepdims=True)
    v = vec / jnp.maximum(r, 1e-12)
    x, y, z = v[..., 0], v[..., 1], v[..., 2]
    s3 = math.sqrt(3.0)
    sh0 = jnp.ones_like(x)[..., None]
    sh1 = s3 * jnp.stack([x, y, z], axis=-1)
    sh2 = math.sqrt(5.0) * jnp.stack(
        [s3 * x * z,
         s3 * x * y,
         y * y - 0.5 * (x * x + z * z),
         s3 * y * z,
         0.5 * s3 * (z * z - x * x)], axis=-1)
    return jnp.concatenate([sh0, sh1, sh2], axis=-1).astype(jnp.float32)


def kernel(node_feature, edge_vec, edge_feature, edge_index, node_w, node_b,
           w1_0, b1_0, w2_0, b2_0, E1_0, E2_0, TB_0, E4_0,
           w1_1, b1_1, w2_1, b2_1, E1_1, E2_1, TB_1, E4_1,
           w1_2, b1_2, w2_2, b2_2, E1_2, E2_2, TB_2, E4_2):
    layers = [
        (w1_0, b1_0, w2_0, b2_0, E1_0, E2_0, TB_0),
        (w1_1, b1_1, w2_1, b2_1, E1_1, E2_1, TB_1),
        (w1_2, b1_2, w2_2, b2_2, E1_2, E2_2, TB_2),
    ]
    edge_src, edge_dst = edge_index[0], edge_index[1]
    n_nodes = node_feature.shape[0]
    n_edges = edge_vec.shape[0]
    e_pad = _round_up(n_edges, _TILE_E)
    pad = e_pad - n_edges

    sh = _sph_harm(edge_vec)
    if pad:
        sh = jnp.pad(sh, ((0, pad), (0, 0)))
        edge_feature = jnp.pad(edge_feature, ((0, pad), (0, 0)))
        edge_dst = jnp.pad(edge_dst, (0, pad))

    nf = _node_linear(node_feature, node_w, node_b)

    cnt = jnp.zeros((n_nodes,), jnp.float32).at[edge_src].add(1.0)
    inv = 1.0 / jnp.maximum(cnt, 1.0)

    bf16 = jnp.bfloat16
    sh_b = sh
    ea_b = edge_feature
    dst3 = edge_dst.astype(jnp.int32).reshape(e_pad // _TILE_E, 1, _TILE_E)

    for i, (w1, b1, w2, b2, e1, e2, tb) in enumerate(layers):
        st = _STRUCT[i]
        c_extra = st["c_pad"] - st["C"]
        tbc = jnp.take(tb, jnp.asarray(st["idx_tb"]), axis=1)
        w2c = jnp.take(w2, jnp.asarray(st["idx_p"]), axis=1)
        b2c = jnp.take(b2, jnp.asarray(st["idx_p"]), axis=1)
        if c_extra:
            tbc = jnp.pad(tbc, ((0, 0), (0, c_extra)))
            w2c = jnp.pad(w2c, ((0, 0), (0, c_extra)))
            b2c = jnp.pad(b2c, ((0, 0), (0, c_extra)))
        e4c = jnp.asarray(st["e4c"].astype(np.float32))

        nf3 = nf[:, None, :]
        tbc_b = tbc.astype(bf16)
        n_tiles = e_pad // _TILE_E
        half_t = n_tiles // 2
        half_e = half_t * _TILE_E
        parts = []
        for lo, hi, tlo, thi in ((0, half_e, 0, half_t),
                                 (half_e, e_pad, half_t, n_tiles)):
            tp_c = _tp_layer(nf3, dst3[tlo:thi], sh_b[lo:hi], ea_b[lo:hi],
                             w1, b1, w2c, b2c, e1, e2, tbc_b, e4c,
                             st["do"], _TILE_E)
            src_c = edge_src[lo:min(hi, n_edges)]
            parts.append(jnp.zeros((n_nodes, st["do"]), jnp.float32)
                         .at[src_c].add(tp_c[:src_c.shape[0]]))
        out = (parts[0] + parts[1]) * inv[:, None]
        if i == 0:
            out = out + jnp.pad(nf, ((0, 0), (0, st["do"] - nf.shape[1])))
        nf = out
    return nf


# speedup vs baseline: 1.8243x; 1.0108x over previous
import functools

import numpy as np
import jax
import jax.numpy as jnp
from jax.experimental import pallas as pl
from jax.experimental.pallas import tpu as pltpu


def _parse(s):
    out = []
    for tok in s.split("+"):
        tok = tok.strip()
        mul, ir = tok.split("x")
        out.append((int(mul), int(ir[:-1]), 1 if ir[-1] == "e" else -1))
    return out


def _dim(irreps):
    return sum(mul * (2 * l + 1) for mul, l, _ in irreps)


def _round_up(x, m):
    return ((x + m - 1) // m) * m


def _compact_structure(ir1_s, ir2_s, iro_s):
    ir1, ir2, iro = _parse(ir1_s), _parse(ir2_s), _parse(iro_s)
    d1, d2, do = _dim(ir1), _dim(ir2), _dim(iro)

    offo, o = [], 0
    for mul, l, _ in iro:
        offo.append(o)
        o += mul * (2 * l + 1)

    instructions = []
    for i1, (m1, l1, p1) in enumerate(ir1):
        for i2, (m2, l2, p2) in enumerate(ir2):
            for io, (mo, lo, po) in enumerate(iro):
                if po == p1 * p2 and abs(l1 - l2) <= lo <= l1 + l2:
                    instructions.append((i1, i2, io))

    P = sum(ir1[i1][0] * ir2[i2][0] * iro[io][0] for i1, i2, io in instructions)
    p_pad = _round_up(P, 128)

    idx_tb, idx_p, idx_k = [], [], []
    p_off = 0
    for i1, i2, io in instructions:
        mul1 = ir1[i1][0]
        mul2 = ir2[i2][0]
        mulo, lo, _ = iro[io]
        ddo = 2 * lo + 1
        for u in range(mul1):
            for v in range(mul2):
                for w in range(mulo):
                    p = p_off + (u * mul2 + v) * mulo + w
                    k0 = offo[io] + w * ddo
                    for mo in range(ddo):
                        idx_tb.append((k0 + mo) * p_pad + p)
                        idx_p.append(p)
                        idx_k.append(k0 + mo)
        p_off += mul1 * mul2 * mulo

    order = np.argsort(np.asarray(idx_tb), kind="stable")
    idx_tb = np.asarray(idx_tb, np.int32)[order]
    idx_p = np.asarray(idx_p, np.int32)[order]
    idx_k = np.asarray(idx_k, np.int32)[order]

    C = idx_tb.shape[0]
    c_pad = _round_up(C, 128)
    e4c = np.zeros((c_pad, do), np.float32)
    e4c[np.arange(C), idx_k] = 1.0
    return dict(idx_tb=idx_tb, idx_p=idx_p, e4c=e4c, C=C, c_pad=c_pad,
                d1=d1, d2=d2, do=do)


_SEQ = [
    "16x0e",
    "16x0e + 2x1o + 2x2e",
    "16x0e + 2x1o + 2x1e + 2x2e + 2x2o",
    "1x0e + 1x0o + 1x1e + 1x1o + 1x2e + 1x2o + 1x3e + 1x3o",
]
_SH_IRREPS = "1x0e + 1x1o + 1x2e"
_STRUCT = [_compact_structure(_SEQ[i], _SH_IRREPS, _SEQ[i + 1]) for i in range(3)]

_TILE_E = 512


def _node_linear_kernel(x_ref, w_ref, b_ref, o_ref):
    o_ref[...] = (jnp.dot(x_ref[...], w_ref[...],
                          preferred_element_type=jnp.float32) + b_ref[...])


def _node_linear(x, w, b):
    n, din = x.shape
    dout = w.shape[1]
    tile = 2048
    while n % tile:
        tile //= 2
    return pl.pallas_call(
        _node_linear_kernel,
        out_shape=jax.ShapeDtypeStruct((n, dout), jnp.float32),
        grid=(n // tile,),
        in_specs=[pl.BlockSpec((tile, din), lambda i: (i, 0)),
                  pl.BlockSpec((din, dout), lambda i: (0, 0)),
                  pl.BlockSpec((1, dout), lambda i: (0, 0))],
        out_specs=pl.BlockSpec((tile, dout), lambda i: (i, 0)),
        compiler_params=pltpu.CompilerParams(
            dimension_semantics=("parallel",)),
    )(x, w, b)


def _tp_kernel(tile_e, dst_ref, nf_ref, sh_ref, ea_ref,
               w1_ref, b1_ref, w2c_ref, b2c_ref,
               e1_ref, e2_ref, tbc_ref, e4c_ref,
               o_ref, x1t_ref):
    f32 = jnp.float32
    bf16 = jnp.bfloat16

    for mi in range(tile_e):
        x1t_ref[mi, :] = nf_ref[dst_ref[0, 0, mi], 0, :]

    h = jnp.dot(ea_ref[...], w1_ref[...], preferred_element_type=f32) + b1_ref[...]
    h = jnp.where(h > 20.0, h, jnp.log1p(jnp.exp(jnp.minimum(h, 20.0))))
    we = jnp.dot(h.astype(bf16), w2c_ref[...],
                 preferred_element_type=f32) + b2c_ref[...]

    x1e = jnp.dot(x1t_ref[...].astype(bf16), e1_ref[...],
                  preferred_element_type=f32)
    she = jnp.dot(sh_ref[...], e2_ref[...], preferred_element_type=f32)
    z = x1e * she

    y = jnp.dot(z.astype(bf16), tbc_ref[...], preferred_element_type=f32)
    o_ref[...] = jnp.dot((we * y).astype(bf16), e4c_ref[...],
                         preferred_element_type=f32)


def _tp_layer(nf, dst3, sh, ea, w1, b1, w2c, b2c, e1, e2, tbc, e4c, do, tile_e):
    e_pad = sh.shape[0]
    d2 = sh.shape[1]
    ed = ea.shape[1]
    n_nodes, _, d1 = nf.shape

    def edge_map(i):
        return (i, 0)

    def const_map(i):
        return (0, 0)

    in_specs = [
        pl.BlockSpec((1, 1, tile_e), lambda i: (i, 0, 0),
                     memory_space=pltpu.SMEM),
        pl.BlockSpec((n_nodes, 1, d1), lambda i: (0, 0, 0)),
        pl.BlockSpec((tile_e, d2), edge_map),
        pl.BlockSpec((tile_e, ed), edge_map),
        pl.BlockSpec(w1.shape, const_map),
        pl.BlockSpec(b1.shape, const_map),
        pl.BlockSpec(w2c.shape, const_map),
        pl.BlockSpec(b2c.shape, const_map),
        pl.BlockSpec(e1.shape, const_map),
        pl.BlockSpec(e2.shape, const_map),
        pl.BlockSpec(tbc.shape, const_map),
        pl.BlockSpec(e4c.shape, const_map),
    ]
    return pl.pallas_call(
        functools.partial(_tp_kernel, tile_e),
        out_shape=jax.ShapeDtypeStruct((e_pad, do), jnp.float32),
        grid=(e_pad // tile_e,),
        in_specs=in_specs,
        out_specs=pl.BlockSpec((tile_e, do), edge_map),
        scratch_shapes=[pltpu.VMEM((tile_e, d1), jnp.float32)],
        compiler_params=pltpu.CompilerParams(
            dimension_semantics=("parallel",),
            vmem_limit_bytes=96 * 1024 * 1024),
    )(dst3, nf, sh, ea, w1, b1, w2c, b2c, e1, e2, tbc, e4c)


def _sph_harm(vec):
    import math
    r = jnp.linalg.norm(vec, axis=-1, keepdims=True)
    v = vec / jnp.maximum(r, 1e-12)
    x, y, z = v[..., 0], v[..., 1], v[..., 2]
    s3 = math.sqrt(3.0)
    sh0 = jnp.ones_like(x)[..., None]
    sh1 = s3 * jnp.stack([x, y, z], axis=-1)
    sh2 = math.sqrt(5.0) * jnp.stack(
        [s3 * x * z,
         s3 * x * y,
         y * y - 0.5 * (x * x + z * z),
         s3 * y * z,
         0.5 * s3 * (z * z - x * x)], axis=-1)
    return jnp.concatenate([sh0, sh1, sh2], axis=-1).astype(jnp.float32)


def kernel(node_feature, edge_vec, edge_feature, edge_index, node_w, node_b,
           w1_0, b1_0, w2_0, b2_0, E1_0, E2_0, TB_0, E4_0,
           w1_1, b1_1, w2_1, b2_1, E1_1, E2_1, TB_1, E4_1,
           w1_2, b1_2, w2_2, b2_2, E1_2, E2_2, TB_2, E4_2):
    layers = [
        (w1_0, b1_0, w2_0, b2_0, E1_0, E2_0, TB_0),
        (w1_1, b1_1, w2_1, b2_1, E1_1, E2_1, TB_1),
        (w1_2, b1_2, w2_2, b2_2, E1_2, E2_2, TB_2),
    ]
    edge_src, edge_dst = edge_index[0], edge_index[1]
    n_nodes = node_feature.shape[0]
    n_edges = edge_vec.shape[0]
    e_pad = _round_up(n_edges, _TILE_E)
    pad = e_pad - n_edges

    sh = _sph_harm(edge_vec)
    if pad:
        sh = jnp.pad(sh, ((0, pad), (0, 0)))
        edge_feature = jnp.pad(edge_feature, ((0, pad), (0, 0)))
        edge_dst = jnp.pad(edge_dst, (0, pad))

    nf = _node_linear(node_feature, node_w, node_b)

    cnt = jnp.zeros((n_nodes,), jnp.float32).at[edge_src].add(1.0)
    inv = 1.0 / jnp.maximum(cnt, 1.0)

    bf16 = jnp.bfloat16
    sh_b = sh.astype(bf16)
    ea_b = edge_feature.astype(bf16)
    dst3 = edge_dst.astype(jnp.int32).reshape(e_pad // _TILE_E, 1, _TILE_E)

    for i, (w1, b1, w2, b2, e1, e2, tb) in enumerate(layers):
        st = _STRUCT[i]
        c_extra = st["c_pad"] - st["C"]
        tbc = jnp.take(tb, jnp.asarray(st["idx_tb"]), axis=1)
        w2c = jnp.take(w2, jnp.asarray(st["idx_p"]), axis=1)
        b2c = jnp.take(b2, jnp.asarray(st["idx_p"]), axis=1)
        if c_extra:
            tbc = jnp.pad(tbc, ((0, 0), (0, c_extra)))
            w2c = jnp.pad(w2c, ((0, 0), (0, c_extra)))
            b2c = jnp.pad(b2c, ((0, 0), (0, c_extra)))
        e4c = jnp.asarray(st["e4c"].astype(np.float32))

        nf3 = nf[:, None, :]
        tbc_b = tbc.astype(bf16)
        n_tiles = e_pad // _TILE_E
        half_t = n_tiles // 2
        half_e = half_t * _TILE_E
        parts = []
        for lo, hi, tlo, thi in ((0, half_e, 0, half_t),
                                 (half_e, e_pad, half_t, n_tiles)):
            tp_c = _tp_layer(nf3, dst3[tlo:thi], sh_b[lo:hi], ea_b[lo:hi],
                             w1.astype(bf16), b1, w2c.astype(bf16), b2c,
                             e1.astype(bf16), e2.astype(bf16), tbc_b,
                             e4c.astype(bf16), st["do"], _TILE_E)
            src_c = edge_src[lo:min(hi, n_edges)]
            parts.append(jnp.zeros((n_nodes, st["do"]), jnp.float32)
                         .at[src_c].add(tp_c[:src_c.shape[0]]))
        out = (parts[0] + parts[1]) * inv[:, None]
        if i == 0:
            out = out + jnp.pad(nf, ((0, 0), (0, st["do"] - nf.shape[1])))
        nf = out
    return nf


# speedup vs baseline: 1.9728x; 1.0814x over previous
import functools

import numpy as np
import jax
import jax.numpy as jnp
from jax.experimental import pallas as pl
from jax.experimental.pallas import tpu as pltpu


def _parse(s):
    out = []
    for tok in s.split("+"):
        tok = tok.strip()
        mul, ir = tok.split("x")
        out.append((int(mul), int(ir[:-1]), 1 if ir[-1] == "e" else -1))
    return out


def _dim(irreps):
    return sum(mul * (2 * l + 1) for mul, l, _ in irreps)


def _round_up(x, m):
    return ((x + m - 1) // m) * m


def _compact_structure(ir1_s, ir2_s, iro_s):
    ir1, ir2, iro = _parse(ir1_s), _parse(ir2_s), _parse(iro_s)
    d1, d2, do = _dim(ir1), _dim(ir2), _dim(iro)

    offo, o = [], 0
    for mul, l, _ in iro:
        offo.append(o)
        o += mul * (2 * l + 1)

    instructions = []
    for i1, (m1, l1, p1) in enumerate(ir1):
        for i2, (m2, l2, p2) in enumerate(ir2):
            for io, (mo, lo, po) in enumerate(iro):
                if po == p1 * p2 and abs(l1 - l2) <= lo <= l1 + l2:
                    instructions.append((i1, i2, io))

    P = sum(ir1[i1][0] * ir2[i2][0] * iro[io][0] for i1, i2, io in instructions)
    p_pad = _round_up(P, 128)

    idx_tb, idx_p, idx_k = [], [], []
    p_off = 0
    for i1, i2, io in instructions:
        mul1 = ir1[i1][0]
        mul2 = ir2[i2][0]
        mulo, lo, _ = iro[io]
        ddo = 2 * lo + 1
        for u in range(mul1):
            for v in range(mul2):
                for w in range(mulo):
                    p = p_off + (u * mul2 + v) * mulo + w
                    k0 = offo[io] + w * ddo
                    for mo in range(ddo):
                        idx_tb.append((k0 + mo) * p_pad + p)
                        idx_p.append(p)
                        idx_k.append(k0 + mo)
        p_off += mul1 * mul2 * mulo

    order = np.argsort(np.asarray(idx_tb), kind="stable")
    idx_tb = np.asarray(idx_tb, np.int32)[order]
    idx_p = np.asarray(idx_p, np.int32)[order]
    idx_k = np.asarray(idx_k, np.int32)[order]

    C = idx_tb.shape[0]
    c_pad = _round_up(C, 128)
    e4c = np.zeros((c_pad, do), np.float32)
    e4c[np.arange(C), idx_k] = 1.0
    return dict(idx_tb=idx_tb, idx_p=idx_p, e4c=e4c, C=C, c_pad=c_pad,
                d1=d1, d2=d2, do=do)


_SEQ = [
    "16x0e",
    "16x0e + 2x1o + 2x2e",
    "16x0e + 2x1o + 2x1e + 2x2e + 2x2o",
    "1x0e + 1x0o + 1x1e + 1x1o + 1x2e + 1x2o + 1x3e + 1x3o",
]
_SH_IRREPS = "1x0e + 1x1o + 1x2e"
_STRUCT = [_compact_structure(_SEQ[i], _SH_IRREPS, _SEQ[i + 1]) for i in range(3)]

_TILE_E = 1024


def _node_linear_kernel(x_ref, w_ref, b_ref, o_ref):
    o_ref[...] = (jnp.dot(x_ref[...], w_ref[...],
                          preferred_element_type=jnp.float32) + b_ref[...])


def _node_linear(x, w, b):
    n, din = x.shape
    dout = w.shape[1]
    tile = 2048
    while n % tile:
        tile //= 2
    return pl.pallas_call(
        _node_linear_kernel,
        out_shape=jax.ShapeDtypeStruct((n, dout), jnp.float32),
        grid=(n // tile,),
        in_specs=[pl.BlockSpec((tile, din), lambda i: (i, 0)),
                  pl.BlockSpec((din, dout), lambda i: (0, 0)),
                  pl.BlockSpec((1, dout), lambda i: (0, 0))],
        out_specs=pl.BlockSpec((tile, dout), lambda i: (i, 0)),
        compiler_params=pltpu.CompilerParams(
            dimension_semantics=("parallel",)),
    )(x, w, b)


def _tp_kernel(tile_e, dst_ref, nf_ref, sh_ref, ea_ref,
               w1_ref, b1_ref, w2c_ref, b2c_ref,
               e1_ref, e2_ref, tbc_ref, e4c_ref,
               o_ref, x1t_ref):
    f32 = jnp.float32
    bf16 = jnp.bfloat16

    for mi in range(tile_e):
        x1t_ref[mi, :] = nf_ref[dst_ref[0, 0, mi], 0, :]

    h = jnp.dot(ea_ref[...], w1_ref[...], preferred_element_type=f32) + b1_ref[...]
    h = jnp.where(h > 20.0, h, jnp.log1p(jnp.exp(jnp.minimum(h, 20.0))))
    we = jnp.dot(h.astype(bf16), w2c_ref[...],
                 preferred_element_type=f32) + b2c_ref[...]

    x1e = jnp.dot(x1t_ref[...].astype(bf16), e1_ref[...],
                  preferred_element_type=f32)
    she = jnp.dot(sh_ref[...], e2_ref[...], preferred_element_type=f32)
    z = x1e * she

    y = jnp.dot(z.astype(bf16), tbc_ref[...], preferred_element_type=f32)
    o_ref[...] = jnp.dot((we * y).astype(bf16), e4c_ref[...],
                         preferred_element_type=f32)


def _tp_layer(nf, dst3, sh, ea, w1, b1, w2c, b2c, e1, e2, tbc, e4c, do, tile_e):
    e_pad = sh.shape[0]
    d2 = sh.shape[1]
    ed = ea.shape[1]
    n_nodes, _, d1 = nf.shape

    def edge_map(i):
        return (i, 0)

    def const_map(i):
        return (0, 0)

    in_specs = [
        pl.BlockSpec((1, 1, tile_e), lambda i: (i, 0, 0),
                     memory_space=pltpu.SMEM),
        pl.BlockSpec((n_nodes, 1, d1), lambda i: (0, 0, 0)),
        pl.BlockSpec((tile_e, d2), edge_map),
        pl.BlockSpec((tile_e, ed), edge_map),
        pl.BlockSpec(w1.shape, const_map),
        pl.BlockSpec(b1.shape, const_map),
        pl.BlockSpec(w2c.shape, const_map),
        pl.BlockSpec(b2c.shape, const_map),
        pl.BlockSpec(e1.shape, const_map),
        pl.BlockSpec(e2.shape, const_map),
        pl.BlockSpec(tbc.shape, const_map),
        pl.BlockSpec(e4c.shape, const_map),
    ]
    return pl.pallas_call(
        functools.partial(_tp_kernel, tile_e),
        out_shape=jax.ShapeDtypeStruct((e_pad, do), jnp.float32),
        grid=(e_pad // tile_e,),
        in_specs=in_specs,
        out_specs=pl.BlockSpec((tile_e, do), edge_map),
        scratch_shapes=[pltpu.VMEM((tile_e, d1), jnp.float32)],
        compiler_params=pltpu.CompilerParams(
            dimension_semantics=("parallel",),
            vmem_limit_bytes=96 * 1024 * 1024),
    )(dst3, nf, sh, ea, w1, b1, w2c, b2c, e1, e2, tbc, e4c)


def _sph_harm(vec):
    import math
    r = jnp.linalg.norm(vec, axis=-1, keepdims=True)
    v = vec / jnp.maximum(r, 1e-12)
    x, y, z = v[..., 0], v[..., 1], v[..., 2]
    s3 = math.sqrt(3.0)
    sh0 = jnp.ones_like(x)[..., None]
    sh1 = s3 * jnp.stack([x, y, z], axis=-1)
    sh2 = math.sqrt(5.0) * jnp.stack(
        [s3 * x * z,
         s3 * x * y,
         y * y - 0.5 * (x * x + z * z),
         s3 * y * z,
         0.5 * s3 * (z * z - x * x)], axis=-1)
    return jnp.concatenate([sh0, sh1, sh2], axis=-1).astype(jnp.float32)


def kernel(node_feature, edge_vec, edge_feature, edge_index, node_w, node_b,
           w1_0, b1_0, w2_0, b2_0, E1_0, E2_0, TB_0, E4_0,
           w1_1, b1_1, w2_1, b2_1, E1_1, E2_1, TB_1, E4_1,
           w1_2, b1_2, w2_2, b2_2, E1_2, E2_2, TB_2, E4_2):
    layers = [
        (w1_0, b1_0, w2_0, b2_0, E1_0, E2_0, TB_0),
        (w1_1, b1_1, w2_1, b2_1, E1_1, E2_1, TB_1),
        (w1_2, b1_2, w2_2, b2_2, E1_2, E2_2, TB_2),
    ]
    edge_src, edge_dst = edge_index[0], edge_index[1]
    n_nodes = node_feature.shape[0]
    n_edges = edge_vec.shape[0]
    e_pad = _round_up(n_edges, _TILE_E)
    pad = e_pad - n_edges

    sh = _sph_harm(edge_vec)
    if pad:
        sh = jnp.pad(sh, ((0, pad), (0, 0)))
        edge_feature = jnp.pad(edge_feature, ((0, pad), (0, 0)))
        edge_dst = jnp.pad(edge_dst, (0, pad))

    nf = _node_linear(node_feature, node_w, node_b)

    cnt = jnp.zeros((n_nodes,), jnp.float32).at[edge_src].add(1.0)
    inv = 1.0 / jnp.maximum(cnt, 1.0)

    bf16 = jnp.bfloat16
    sh_b = sh.astype(bf16)
    ea_b = edge_feature.astype(bf16)
    dst3 = edge_dst.astype(jnp.int32).reshape(e_pad // _TILE_E, 1, _TILE_E)

    for i, (w1, b1, w2, b2, e1, e2, tb) in enumerate(layers):
        st = _STRUCT[i]
        c_extra = st["c_pad"] - st["C"]
        tbc = jnp.take(tb, jnp.asarray(st["idx_tb"]), axis=1)
        w2c = jnp.take(w2, jnp.asarray(st["idx_p"]), axis=1)
        b2c = jnp.take(b2, jnp.asarray(st["idx_p"]), axis=1)
        if c_extra:
            tbc = jnp.pad(tbc, ((0, 0), (0, c_extra)))
            w2c = jnp.pad(w2c, ((0, 0), (0, c_extra)))
            b2c = jnp.pad(b2c, ((0, 0), (0, c_extra)))
        e4c = jnp.asarray(st["e4c"].astype(np.float32))

        nf3 = nf[:, None, :]
        tbc_b = tbc.astype(bf16)
        n_tiles = e_pad // _TILE_E
        half_t = n_tiles // 2
        half_e = half_t * _TILE_E
        parts = []
        for lo, hi, tlo, thi in ((0, half_e, 0, half_t),
                                 (half_e, e_pad, half_t, n_tiles)):
            tp_c = _tp_layer(nf3, dst3[tlo:thi], sh_b[lo:hi], ea_b[lo:hi],
                             w1.astype(bf16), b1, w2c.astype(bf16), b2c,
                             e1.astype(bf16), e2.astype(bf16), tbc_b,
                             e4c.astype(bf16), st["do"], _TILE_E)
            src_c = edge_src[lo:min(hi, n_edges)]
            parts.append(jnp.zeros((n_nodes, st["do"]), jnp.float32)
                         .at[src_c].add(tp_c[:src_c.shape[0]]))
        out = (parts[0] + parts[1]) * inv[:, None]
        if i == 0:
            out = out + jnp.pad(nf, ((0, 0), (0, st["do"] - nf.shape[1])))
        nf = out
    return nf
